# baseline (speedup 1.0000x reference)
import jax
import jax.numpy as jnp
from jax import lax
from jax.experimental import pallas as pl
from jax.experimental.pallas import tpu as pltpu
from jax.experimental.pallas import tpu_sc as plsc

B, S, D = 16384, 200, 32
N = B * S
NC, NS = 2, 16
NW = NC * NS
PER_W = N // NW
K = 512
NBLK = PER_W // K
G = K // 16
NH, ND, NM = 24, 7, 12
NCOMB = NH * ND * NM


def _body(hours_ref, days_ref, months_ref, ht_ref, dt_ref, mt_ref, out_ref,
          ht_v, dt_v, mt_v, comb,
          h_idx0, d_idx0, m_idx0, obuf0, h_idx1, d_idx1, m_idx1, obuf1,
          sem_in0, sem_in1, sem_out0, sem_out1):
    wid = lax.axis_index("s") * NC + lax.axis_index("c")
    w_base = wid * PER_W

    pltpu.sync_copy(ht_ref, ht_v)
    pltpu.sync_copy(dt_ref, dt_v)
    pltpu.sync_copy(mt_ref, mt_v)

    def h_loop(h, carry):
        h0 = ht_v[pl.ds(h * D, 16)]
        h1 = ht_v[pl.ds(h * D + 16, 16)]
        for d in range(ND):
            hd0 = h0 + dt_v[pl.ds(d * D, 16)]
            hd1 = h1 + dt_v[pl.ds(d * D + 16, 16)]
            for m in range(NM):
                r = ((h * ND + d) * NM + m) * D
                comb[pl.ds(r, 16)] = hd0 + mt_v[pl.ds(m * D, 16)]
                comb[pl.ds(r + 16, 16)] = hd1 + mt_v[pl.ds(m * D + 16, 16)]
        return carry

    lax.fori_loop(0, NH, h_loop, 0)

    iota16 = lax.iota(jnp.int32, 16)
    slots = ((h_idx0, d_idx0, m_idx0, obuf0, sem_in0, sem_out0),
             (h_idx1, d_idx1, m_idx1, obuf1, sem_in1, sem_out1))

    def issue_in(b, slot):
        h_i, d_i, m_i, _, s_in, _ = slot
        base = w_base + b * K
        pltpu.async_copy(hours_ref.at[pl.ds(base, K)], h_i, s_in)
        pltpu.async_copy(days_ref.at[pl.ds(base, K)], d_i, s_in)
        pltpu.async_copy(months_ref.at[pl.ds(base, K)], m_i, s_in)

    def wait_in(b, slot):
        h_i, d_i, m_i, _, s_in, _ = slot
        base = w_base + b * K
        pltpu.make_async_copy(hours_ref.at[pl.ds(base, K)], h_i, s_in).wait()
        pltpu.make_async_copy(days_ref.at[pl.ds(base, K)], d_i, s_in).wait()
        pltpu.make_async_copy(months_ref.at[pl.ds(base, K)], m_i, s_in).wait()

    def issue_out(b, slot):
        _, _, _, ob, _, s_out = slot
        base = w_base + b * K
        pltpu.async_copy(ob, out_ref.at[pl.ds(base * D, K * D)], s_out)

    def wait_out(b, slot):
        _, _, _, ob, _, s_out = slot
        base = w_base + b * K
        pltpu.make_async_copy(ob, out_ref.at[pl.ds(base * D, K * D)],
                              s_out).wait()

    def compute(slot):
        h_i, d_i, m_i, ob, _, _ = slot

        @plsc.parallel_loop(0, G, step=1, unroll=2)
        def group(g):
            h = h_i[pl.ds(g * 16, 16)]
            d = d_i[pl.ds(g * 16, 16)]
            m = m_i[pl.ds(g * 16, 16)]
            a = ((h * (ND * NM) + d * NM) + m) * D
            r = (g * 16 + iota16) * D
            for j0 in range(0, D, 8):
                vals = [plsc.load_gather(comb, [a + j])
                        for j in range(j0, j0 + 8)]
                for j in range(j0, j0 + 8):
                    plsc.store_scatter(ob, [r + j], vals[j - j0])

    issue_in(0, slots[0])

    def block2(i, carry):
        for half, slot in ((0, slots[0]), (1, slots[1])):
            b = 2 * i + half
            nxt = slots[1 - half]

            @pl.when(b + 1 < NBLK)
            def _():
                issue_in(b + 1, nxt)

            wait_in(b, slot)

            @pl.when(b >= 2)
            def _():
                wait_out(b - 2, slot)

            compute(slot)
            issue_out(b, slot)
        return carry

    lax.fori_loop(0, NBLK // 2, block2, 0)
    wait_out(NBLK - 2, slots[0])
    wait_out(NBLK - 1, slots[1])


@jax.jit
def _run(hours2, days2, months2, ht_f, dt_f, mt_f):
    mesh = plsc.VectorSubcoreMesh(core_axis_name="c", subcore_axis_name="s")
    kern = pl.kernel(
        _body,
        out_type=jax.ShapeDtypeStruct((N * D,), jnp.float32),
        mesh=mesh,
        scratch_types=[
            pltpu.VMEM((NH * D,), jnp.float32),
            pltpu.VMEM((ND * D,), jnp.float32),
            pltpu.VMEM((NM * D,), jnp.float32),
            pltpu.VMEM((NCOMB * D,), jnp.float32),
            pltpu.VMEM((K,), jnp.int32),
            pltpu.VMEM((K,), jnp.int32),
            pltpu.VMEM((K,), jnp.int32),
            pltpu.VMEM((K * D,), jnp.float32),
            pltpu.VMEM((K,), jnp.int32),
            pltpu.VMEM((K,), jnp.int32),
            pltpu.VMEM((K,), jnp.int32),
            pltpu.VMEM((K * D,), jnp.float32),
            pltpu.SemaphoreType.DMA,
            pltpu.SemaphoreType.DMA,
            pltpu.SemaphoreType.DMA,
            pltpu.SemaphoreType.DMA,
        ],
        compiler_params=pltpu.CompilerParams(
            use_tc_tiling_on_sc=False, needs_layout_passes=False,
            disable_bounds_checks=True),
    )
    return kern(hours2, days2, months2, ht_f, dt_f, mt_f)


def kernel(hours, days, months, hour_table, day_table, month_table):
    hours2 = hours.astype(jnp.int32).reshape(N)
    days2 = days.astype(jnp.int32).reshape(N)
    months2 = months.astype(jnp.int32).reshape(N)
    out = _run(hours2, days2, months2, hour_table.reshape(NH * D),
               day_table.reshape(ND * D), month_table.reshape(NM * D))
    return out.reshape(B, S, D)


# speedup vs baseline: 2.0773x; 2.0773x over previous
import jax
import jax.numpy as jnp
from jax import lax
from jax.experimental import pallas as pl
from jax.experimental.pallas import tpu as pltpu
from jax.experimental.pallas import tpu_sc as plsc

B, S, D = 16384, 200, 32
N = B * S
NC, NS = 2, 16
NW = NC * NS
PER_W = N // NW
K = 512
NBLK = PER_W // K
G = K // 16
GB = 128
NGB = K // GB
NH, ND, NM = 24, 7, 12
NCOMB = NH * ND * NM


def _body(hours_ref, days_ref, months_ref, ht_ref, dt_ref, mt_ref, out_ref,
          ht_v, dt_v, mt_v, comb_v, comb_sh,
          h_idx0, d_idx0, m_idx0, obuf0, cidx0,
          h_idx1, d_idx1, m_idx1, obuf1, cidx1,
          sem_in0, sem_in1, sem_out0, sem_out1, sem_g0, sem_g1):
    wid = lax.axis_index("s") * NC + lax.axis_index("c")
    w_base = wid * PER_W

    @pl.when(lax.axis_index("s") == 0)
    def _():
        pltpu.sync_copy(ht_ref, ht_v)
        pltpu.sync_copy(dt_ref, dt_v)
        pltpu.sync_copy(mt_ref, mt_v)

        def h_loop(h, carry):
            h0 = ht_v[pl.ds(h * D, 16)]
            h1 = ht_v[pl.ds(h * D + 16, 16)]
            for d in range(ND):
                hd0 = h0 + dt_v[pl.ds(d * D, 16)]
                hd1 = h1 + dt_v[pl.ds(d * D + 16, 16)]
                for m in range(NM):
                    r = (h * ND + d) * NM + m
                    comb_v[r, pl.ds(0, 16)] = hd0 + mt_v[pl.ds(m * D, 16)]
                    comb_v[r, pl.ds(16, 16)] = hd1 + mt_v[pl.ds(m * D + 16, 16)]
            return carry

        lax.fori_loop(0, NH, h_loop, 0)
        pltpu.sync_copy(comb_v, comb_sh)

    plsc.subcore_barrier()

    slots = ((h_idx0, d_idx0, m_idx0, obuf0, cidx0, sem_in0, sem_out0, sem_g0),
             (h_idx1, d_idx1, m_idx1, obuf1, cidx1, sem_in1, sem_out1, sem_g1))

    def issue_in(b, slot):
        h_i, d_i, m_i = slot[0], slot[1], slot[2]
        s_in = slot[5]
        base = w_base + b * K
        pltpu.async_copy(hours_ref.at[pl.ds(base, K)], h_i, s_in)
        pltpu.async_copy(days_ref.at[pl.ds(base, K)], d_i, s_in)
        pltpu.async_copy(months_ref.at[pl.ds(base, K)], m_i, s_in)

    def wait_in(b, slot):
        h_i, d_i, m_i = slot[0], slot[1], slot[2]
        s_in = slot[5]
        base = w_base + b * K
        pltpu.make_async_copy(hours_ref.at[pl.ds(base, K)], h_i, s_in).wait()
        pltpu.make_async_copy(days_ref.at[pl.ds(base, K)], d_i, s_in).wait()
        pltpu.make_async_copy(months_ref.at[pl.ds(base, K)], m_i, s_in).wait()

    def issue_out(b, slot):
        ob, s_out = slot[3], slot[6]
        base = w_base + b * K
        pltpu.async_copy(ob, out_ref.at[pl.ds(base, K)], s_out)

    def wait_out(b, slot):
        ob, s_out = slot[3], slot[6]
        base = w_base + b * K
        pltpu.make_async_copy(ob, out_ref.at[pl.ds(base, K)], s_out).wait()

    def compute_cidx(slot):
        h_i, d_i, m_i, ci = slot[0], slot[1], slot[2], slot[4]

        @plsc.parallel_loop(0, G, step=1, unroll=4)
        def group(g):
            h = h_i[pl.ds(g * 16, 16)]
            d = d_i[pl.ds(g * 16, 16)]
            m = m_i[pl.ds(g * 16, 16)]
            ci[pl.ds(g * 16, 16)] = (h * (ND * NM) + d * NM) + m

    def fire_gathers(slot):
        ob, ci, s_g = slot[3], slot[4], slot[7]
        for q in range(NGB):
            pltpu.async_copy(comb_sh.at[ci.at[pl.ds(q * GB, GB)]],
                             ob.at[pl.ds(q * GB, GB)], s_g)

    def drain_gathers(slot):
        ob, ci, s_g = slot[3], slot[4], slot[7]
        for q in range(NGB):
            pltpu.make_async_copy(comb_sh.at[ci.at[pl.ds(q * GB, GB)]],
                                  ob.at[pl.ds(q * GB, GB)], s_g).wait()

    issue_in(0, slots[0])

    def block2(i, carry):
        for half, slot in ((0, slots[0]), (1, slots[1])):
            b = 2 * i + half
            nxt = slots[1 - half]

            @pl.when(b + 1 < NBLK)
            def _():
                issue_in(b + 1, nxt)

            wait_in(b, slot)
            compute_cidx(slot)

            @pl.when(b >= 2)
            def _():
                wait_out(b - 2, slot)

            fire_gathers(slot)
            drain_gathers(slot)
            issue_out(b, slot)
        return carry

    lax.fori_loop(0, NBLK // 2, block2, 0)
    wait_out(NBLK - 2, slots[0])
    wait_out(NBLK - 1, slots[1])


@jax.jit
def _run(hours2, days2, months2, ht_f, dt_f, mt_f):
    mesh = plsc.VectorSubcoreMesh(core_axis_name="c", subcore_axis_name="s")
    kern = pl.kernel(
        _body,
        out_type=jax.ShapeDtypeStruct((N, D), jnp.float32),
        mesh=mesh,
        scratch_types=[
            pltpu.VMEM((NH * D,), jnp.float32),
            pltpu.VMEM((ND * D,), jnp.float32),
            pltpu.VMEM((NM * D,), jnp.float32),
            pltpu.VMEM((NCOMB, D), jnp.float32),
            pltpu.VMEM_SHARED((NCOMB, D), jnp.float32),
            pltpu.VMEM((K,), jnp.int32),
            pltpu.VMEM((K,), jnp.int32),
            pltpu.VMEM((K,), jnp.int32),
            pltpu.VMEM((K, D), jnp.float32),
            pltpu.VMEM((K,), jnp.int32),
            pltpu.VMEM((K,), jnp.int32),
            pltpu.VMEM((K,), jnp.int32),
            pltpu.VMEM((K,), jnp.int32),
            pltpu.VMEM((K, D), jnp.float32),
            pltpu.VMEM((K,), jnp.int32),
            pltpu.SemaphoreType.DMA,
            pltpu.SemaphoreType.DMA,
            pltpu.SemaphoreType.DMA,
            pltpu.SemaphoreType.DMA,
            pltpu.SemaphoreType.DMA,
            pltpu.SemaphoreType.DMA,
        ],
        compiler_params=pltpu.CompilerParams(
            use_tc_tiling_on_sc=False, needs_layout_passes=False,
            disable_bounds_checks=True),
    )
    return kern(hours2, days2, months2, ht_f, dt_f, mt_f)


def kernel(hours, days, months, hour_table, day_table, month_table):
    hours2 = hours.astype(jnp.int32).reshape(N)
    days2 = days.astype(jnp.int32).reshape(N)
    months2 = months.astype(jnp.int32).reshape(N)
    out = _run(hours2, days2, months2, hour_table.reshape(NH * D),
               day_table.reshape(ND * D), month_table.reshape(NM * D))
    return out.reshape(B, S, D)


# speedup vs baseline: 2.0991x; 1.0105x over previous
import jax
import jax.numpy as jnp
from jax import lax
from jax.experimental import pallas as pl
from jax.experimental.pallas import tpu as pltpu
from jax.experimental.pallas import tpu_sc as plsc

B, S, D = 16384, 200, 32
N = B * S
NC, NS = 2, 16
NW = NC * NS
PER_W = N // NW
K = 640
NBLK = PER_W // K
G = K // 16
GB = 128
NGB = K // GB
NH, ND, NM = 24, 7, 12
NCOMB = NH * ND * NM


def _body(hours_ref, days_ref, months_ref, ht_ref, dt_ref, mt_ref, out_ref,
          ht_v, dt_v, mt_v, comb_v, comb_sh,
          h_idx0, d_idx0, m_idx0, obuf0, cidx0,
          h_idx1, d_idx1, m_idx1, obuf1, cidx1,
          sem_in0, sem_in1, sem_out0, sem_out1, sem_g0, sem_g1):
    wid = lax.axis_index("s") * NC + lax.axis_index("c")
    w_base = wid * PER_W

    @pl.when(lax.axis_index("s") == 0)
    def _():
        pltpu.sync_copy(ht_ref, ht_v)
        pltpu.sync_copy(dt_ref, dt_v)
        pltpu.sync_copy(mt_ref, mt_v)

        def h_loop(h, carry):
            h0 = ht_v[pl.ds(h * D, 16)]
            h1 = ht_v[pl.ds(h * D + 16, 16)]
            for d in range(ND):
                hd0 = h0 + dt_v[pl.ds(d * D, 16)]
                hd1 = h1 + dt_v[pl.ds(d * D + 16, 16)]
                for m in range(NM):
                    r = (h * ND + d) * NM + m
                    comb_v[r, pl.ds(0, 16)] = hd0 + mt_v[pl.ds(m * D, 16)]
                    comb_v[r, pl.ds(16, 16)] = hd1 + mt_v[pl.ds(m * D + 16, 16)]
            return carry

        lax.fori_loop(0, NH, h_loop, 0)
        pltpu.sync_copy(comb_v, comb_sh)

    plsc.subcore_barrier()

    slots = ((h_idx0, d_idx0, m_idx0, obuf0, cidx0, sem_in0, sem_out0, sem_g0),
             (h_idx1, d_idx1, m_idx1, obuf1, cidx1, sem_in1, sem_out1, sem_g1))

    def issue_in(b, slot):
        h_i, d_i, m_i = slot[0], slot[1], slot[2]
        s_in = slot[5]
        base = w_base + b * K
        pltpu.async_copy(hours_ref.at[pl.ds(base, K)], h_i, s_in)
        pltpu.async_copy(days_ref.at[pl.ds(base, K)], d_i, s_in)
        pltpu.async_copy(months_ref.at[pl.ds(base, K)], m_i, s_in)

    def wait_in(b, slot):
        h_i, d_i, m_i = slot[0], slot[1], slot[2]
        s_in = slot[5]
        base = w_base + b * K
        pltpu.make_async_copy(hours_ref.at[pl.ds(base, K)], h_i, s_in).wait()
        pltpu.make_async_copy(days_ref.at[pl.ds(base, K)], d_i, s_in).wait()
        pltpu.make_async_copy(months_ref.at[pl.ds(base, K)], m_i, s_in).wait()

    def issue_out(b, slot):
        ob, s_out = slot[3], slot[6]
        base = w_base + b * K
        pltpu.async_copy(ob, out_ref.at[pl.ds(base, K)], s_out)

    def wait_out(b, slot):
        ob, s_out = slot[3], slot[6]
        base = w_base + b * K
        pltpu.make_async_copy(ob, out_ref.at[pl.ds(base, K)], s_out).wait()

    def compute_cidx(slot):
        h_i, d_i, m_i, ci = slot[0], slot[1], slot[2], slot[4]

        @plsc.parallel_loop(0, G, step=1, unroll=4)
        def group(g):
            h = h_i[pl.ds(g * 16, 16)]
            d = d_i[pl.ds(g * 16, 16)]
            m = m_i[pl.ds(g * 16, 16)]
            ci[pl.ds(g * 16, 16)] = (h * (ND * NM) + d * NM) + m

    def fire_gathers(slot):
        ob, ci, s_g = slot[3], slot[4], slot[7]
        for q in range(NGB):
            pltpu.async_copy(comb_sh.at[ci.at[pl.ds(q * GB, GB)]],
                             ob.at[pl.ds(q * GB, GB)], s_g)

    def drain_gathers(slot):
        ob, ci, s_g = slot[3], slot[4], slot[7]
        for q in range(NGB):
            pltpu.make_async_copy(comb_sh.at[ci.at[pl.ds(q * GB, GB)]],
                                  ob.at[pl.ds(q * GB, GB)], s_g).wait()

    issue_in(0, slots[0])

    def block2(i, carry):
        for half, slot in ((0, slots[0]), (1, slots[1])):
            b = 2 * i + half
            nxt = slots[1 - half]

            @pl.when(b + 1 < NBLK)
            def _():
                issue_in(b + 1, nxt)

            wait_in(b, slot)
            compute_cidx(slot)

            @pl.when(b >= 2)
            def _():
                wait_out(b - 2, slot)

            fire_gathers(slot)
            drain_gathers(slot)
            issue_out(b, slot)
        return carry

    lax.fori_loop(0, NBLK // 2, block2, 0)
    wait_out(NBLK - 2, slots[0])
    wait_out(NBLK - 1, slots[1])


@jax.jit
def _run(hours2, days2, months2, ht_f, dt_f, mt_f):
    mesh = plsc.VectorSubcoreMesh(core_axis_name="c", subcore_axis_name="s")
    kern = pl.kernel(
        _body,
        out_type=jax.ShapeDtypeStruct((N, D), jnp.float32),
        mesh=mesh,
        scratch_types=[
            pltpu.VMEM((NH * D,), jnp.float32),
            pltpu.VMEM((ND * D,), jnp.float32),
            pltpu.VMEM((NM * D,), jnp.float32),
            pltpu.VMEM((NCOMB, D), jnp.float32),
            pltpu.VMEM_SHARED((NCOMB, D), jnp.float32),
            pltpu.VMEM((K,), jnp.int32),
            pltpu.VMEM((K,), jnp.int32),
            pltpu.VMEM((K,), jnp.int32),
            pltpu.VMEM((K, D), jnp.float32),
            pltpu.VMEM((K,), jnp.int32),
            pltpu.VMEM((K,), jnp.int32),
            pltpu.VMEM((K,), jnp.int32),
            pltpu.VMEM((K,), jnp.int32),
            pltpu.VMEM((K, D), jnp.float32),
            pltpu.VMEM((K,), jnp.int32),
            pltpu.SemaphoreType.DMA,
            pltpu.SemaphoreType.DMA,
            pltpu.SemaphoreType.DMA,
            pltpu.SemaphoreType.DMA,
            pltpu.SemaphoreType.DMA,
            pltpu.SemaphoreType.DMA,
        ],
        compiler_params=pltpu.CompilerParams(
            use_tc_tiling_on_sc=False, needs_layout_passes=False,
            disable_bounds_checks=True),
    )
    return kern(hours2, days2, months2, ht_f, dt_f, mt_f)


def kernel(hours, days, months, hour_table, day_table, month_table):
    hours2 = hours.astype(jnp.int32).reshape(N)
    days2 = days.astype(jnp.int32).reshape(N)
    months2 = months.astype(jnp.int32).reshape(N)
    out = _run(hours2, days2, months2, hour_table.reshape(NH * D),
               day_table.reshape(ND * D), month_table.reshape(NM * D))
    return out.reshape(B, S, D)
